# baseline (speedup 1.0000x reference)
import jax
import jax.numpy as jnp
from jax import lax
from jax.experimental import pallas as pl
from jax.experimental.pallas import tpu as pltpu

_ROWS = 16384
_COLS = 200
_RB = 8
_GRID = _COLS // _RB


def _tc_body(w_ref, x_ref, out_ref, acc_ref, sem):
    i = pl.program_id(0)
    b = lax.rem(i, 2)

    def dma(step, buf):
        return pltpu.make_async_copy(
            acc_ref.at[buf], out_ref.at[pl.ds(step * _RB, _RB)], sem)

    @pl.when(i > 0)
    def _():
        dma(i - 1, 1 - b).wait()

    xb = x_ref[...]
    w0 = w_ref[0, 0]
    w1 = w_ref[0, 1]
    w2 = w_ref[0, 2]
    w3 = w_ref[0, 3]
    lo = jnp.where(xb == 1, w1, w0)
    hi = jnp.where(xb == 3, w3, w2)
    acc_ref[b, :, 0, :] = jnp.where(xb >= 2, hi, lo)
    dma(i, b).start()

    @pl.when(i == _GRID - 1)
    def _():
        dma(i, b).wait()


@jax.jit
def kernel(x, weight):
    w_row = weight.reshape(1, 4).astype(jnp.float32)
    xt = x.T
    out_lin = pl.pallas_call(
        _tc_body,
        grid=(_GRID,),
        in_specs=[
            pl.BlockSpec(memory_space=pltpu.SMEM),
            pl.BlockSpec((_RB, _ROWS), lambda i: (i, 0)),
        ],
        out_specs=pl.BlockSpec(memory_space=pl.ANY),
        out_shape=jax.ShapeDtypeStruct((_COLS, 1, _ROWS), jnp.float32),
        scratch_shapes=[
            pltpu.VMEM((2, _RB, 1, _ROWS), jnp.float32),
            pltpu.SemaphoreType.DMA,
        ],
    )(w_row, xt)
    return jnp.transpose(out_lin, (2, 0, 1))


# speedup vs baseline: 1.2988x; 1.2988x over previous
import jax
import jax.numpy as jnp
from jax import lax
from jax.experimental import pallas as pl
from jax.experimental.pallas import tpu as pltpu

_ROWS = 16384
_COLS = 200
_RB = 8
_GRID = _COLS // _RB
_NBUF = 3


def _tc_body(w_ref, x_ref, out_ref, acc_ref, sem):
    i = pl.program_id(0)
    b = lax.rem(i, _NBUF)

    def dma(step, buf, r):
        return pltpu.make_async_copy(
            acc_ref.at[buf, r], out_ref.at[step * _RB + r, 0], sem)

    @pl.when(i >= _NBUF - 1)
    def _():
        j = i - (_NBUF - 1)
        for r in range(_RB):
            dma(j, lax.rem(j, _NBUF), r).wait()

    xb = x_ref[...]
    w0 = w_ref[0, 0]
    w1 = w_ref[0, 1]
    w2 = w_ref[0, 2]
    w3 = w_ref[0, 3]
    lo = jnp.where(xb == 1, w1, w0)
    hi = jnp.where(xb == 3, w3, w2)
    acc_ref[b] = jnp.where(xb >= 2, hi, lo)
    for r in range(_RB):
        dma(i, b, r).start()

    @pl.when(i == _GRID - 1)
    def _():
        for j in range(_GRID - (_NBUF - 1), _GRID):
            for r in range(_RB):
                dma(j, j % _NBUF, r).wait()


@jax.jit
def kernel(x, weight):
    w_row = weight.reshape(1, 4).astype(jnp.float32)
    xt = x.T
    out_lin = pl.pallas_call(
        _tc_body,
        grid=(_GRID,),
        in_specs=[
            pl.BlockSpec(memory_space=pltpu.SMEM),
            pl.BlockSpec((_RB, _ROWS), lambda i: (i, 0)),
        ],
        out_specs=pl.BlockSpec(memory_space=pl.ANY),
        out_shape=jax.ShapeDtypeStruct((_COLS, 1, _ROWS), jnp.float32),
        scratch_shapes=[
            pltpu.VMEM((_NBUF, _RB, _ROWS), jnp.float32),
            pltpu.SemaphoreType.DMA,
        ],
    )(w_row, xt)
    return jnp.transpose(out_lin, (2, 0, 1))


# speedup vs baseline: 1.3107x; 1.0092x over previous
import jax
import jax.numpy as jnp
from jax import lax
from jax.experimental import pallas as pl
from jax.experimental.pallas import tpu as pltpu

_ROWS = 16384
_COLS = 200
_RB = 8
_GRID = _COLS // _RB
_NBUF = 4


def _tc_body(w_ref, x_ref, out_ref, acc_ref, sem):
    i = pl.program_id(0)
    b = lax.rem(i, _NBUF)

    def dma(step, buf, r):
        return pltpu.make_async_copy(
            acc_ref.at[buf, r], out_ref.at[step * _RB + r, 0], sem)

    @pl.when(i >= _NBUF - 1)
    def _():
        j = i - (_NBUF - 1)
        for r in range(_RB):
            dma(j, lax.rem(j, _NBUF), r).wait()

    xb = x_ref[...]
    w0 = w_ref[0, 0]
    w1 = w_ref[0, 1]
    w2 = w_ref[0, 2]
    w3 = w_ref[0, 3]
    lo = jnp.where(xb == 1, w1, w0)
    hi = jnp.where(xb == 3, w3, w2)
    acc_ref[b] = jnp.where(xb >= 2, hi, lo)
    for r in range(_RB):
        dma(i, b, r).start()

    @pl.when(i == _GRID - 1)
    def _():
        for j in range(_GRID - (_NBUF - 1), _GRID):
            for r in range(_RB):
                dma(j, j % _NBUF, r).wait()


@jax.jit
def kernel(x, weight):
    w_row = weight.reshape(1, 4).astype(jnp.float32)
    xt = x.T
    out_lin = pl.pallas_call(
        _tc_body,
        grid=(_GRID,),
        in_specs=[
            pl.BlockSpec(memory_space=pltpu.SMEM),
            pl.BlockSpec((_RB, _ROWS), lambda i: (i, 0)),
        ],
        out_specs=pl.BlockSpec(memory_space=pl.ANY),
        out_shape=jax.ShapeDtypeStruct((_COLS, 1, _ROWS), jnp.float32),
        scratch_shapes=[
            pltpu.VMEM((_NBUF, _RB, _ROWS), jnp.float32),
            pltpu.SemaphoreType.DMA,
        ],
    )(w_row, xt)
    return jnp.transpose(out_lin, (2, 0, 1))
